# baseline (speedup 1.0000x reference)
import functools

import jax
import jax.numpy as jnp
from jax import lax
from jax.experimental import pallas as pl
from jax.experimental.pallas import tpu as pltpu
from jax.experimental.pallas import tpu_sc as plsc

N = 10000
E = 160000
IN_DIM = 256
HIDDEN = 128
OUT_DIM = 64

NC = 2
NS = 16
NW = NC * NS
L = 16

N_PAD = 10240
STRIP = N_PAD // NS
E_PAD = 163840
EPW = E_PAD // NW
CH = 512
CHUNKS = EPW // CH
NEG = -3.0e38

_mesh = plsc.VectorSubcoreMesh(
    core_axis_name="c", subcore_axis_name="s", num_cores=NC, num_subcores=NS
)


def _dup_safe_max(m_ref, idx, val):
    def cond(mask):
        return jnp.any(mask)

    def body(mask):
        cur = plsc.load_gather(m_ref, [idx], mask=mask)
        need = jnp.logical_and(mask, val > cur)
        plsc.store_scatter(m_ref, [idx], val, mask=need)
        back = plsc.load_gather(m_ref, [idx], mask=need)
        return jnp.logical_and(need, back < val)

    lax.while_loop(cond, body, jnp.ones((L,), jnp.bool_))


def _dense_body(x_ref, w0_ref, w1_ref, wa_ref, h_ref, al_ref):
    h1 = jnp.maximum(
        jnp.dot(x_ref[...], w0_ref[...], preferred_element_type=jnp.float32), 0.0
    )
    h2 = jnp.dot(h1, w1_ref[...], preferred_element_type=jnp.float32)
    h_ref[...] = h2
    al_ref[...] = jnp.dot(h2, wa_ref[...], preferred_element_type=jnp.float32)


def _dense(xp, W0, W1, wa):
    return pl.pallas_call(
        _dense_body,
        grid=(10,),
        in_specs=[
            pl.BlockSpec((N_PAD // 10, IN_DIM), lambda i: (i, 0)),
            pl.BlockSpec((IN_DIM, HIDDEN), lambda i: (0, 0)),
            pl.BlockSpec((HIDDEN, OUT_DIM), lambda i: (0, 0)),
            pl.BlockSpec((OUT_DIM, 128), lambda i: (0, 0)),
        ],
        out_specs=[
            pl.BlockSpec((N_PAD // 10, OUT_DIM), lambda i: (i, 0)),
            pl.BlockSpec((N_PAD // 10, 128), lambda i: (i, 0)),
        ],
        out_shape=[
            jax.ShapeDtypeStruct((N_PAD, OUT_DIM), jnp.float32),
            jax.ShapeDtypeStruct((N_PAD, 128), jnp.float32),
        ],
    )(xp, W0, W1, wa)


def _add_body(a_ref, b_ref, o_ref):
    o_ref[...] = a_ref[...] + b_ref[...]


def _add2(a, b, rows, blk):
    return pl.pallas_call(
        _add_body,
        grid=(rows // blk,),
        in_specs=[
            pl.BlockSpec((blk, OUT_DIM), lambda i: (i, 0)),
            pl.BlockSpec((blk, OUT_DIM), lambda i: (i, 0)),
        ],
        out_specs=pl.BlockSpec((blk, OUT_DIM), lambda i: (i, 0)),
        out_shape=jax.ShapeDtypeStruct((rows, OUT_DIM), jnp.float32),
    )(a, b)


def _k2a_body(src_hbm, dst_hbm, als_hbm, ald_hbm, att_hbm, m32_hbm,
              src_v, dst_v, att_v, als_v, ald_v, m_v):
    cid = lax.axis_index("c")
    sid = lax.axis_index("s")
    wid = cid * NS + sid
    base = pl.multiple_of(wid * EPW, 512)
    pltpu.sync_copy(src_hbm.at[pl.ds(base, EPW)], src_v)
    pltpu.sync_copy(dst_hbm.at[pl.ds(base, EPW)], dst_v)
    pltpu.sync_copy(als_hbm, als_v)
    pltpu.sync_copy(ald_hbm, ald_v)

    def init(i, carry):
        m_v[pl.ds(i * L, L)] = jnp.full((L,), NEG, jnp.float32)
        return carry

    lax.fori_loop(0, N_PAD // L, init, 0)

    def edge(i, carry):
        s16 = src_v[pl.ds(i * L, L)]
        d16 = dst_v[pl.ds(i * L, L)]
        av = plsc.load_gather(als_v, [s16]) + plsc.load_gather(ald_v, [d16])
        att = jnp.maximum(av, 0.01 * av)
        att_v[pl.ds(i * L, L)] = att
        _dup_safe_max(m_v, s16, att)
        return carry

    lax.fori_loop(0, EPW // L, edge, 0)
    pltpu.sync_copy(att_v, att_hbm.at[pl.ds(base, EPW)])
    pltpu.sync_copy(m_v, m32_hbm.at[wid])


_k2a = pl.kernel(
    _k2a_body,
    out_type=[
        jax.ShapeDtypeStruct((E_PAD,), jnp.float32),
        jax.ShapeDtypeStruct((NW, N_PAD), jnp.float32),
    ],
    mesh=_mesh,
    compiler_params=pltpu.CompilerParams(needs_layout_passes=False),
    scratch_types=[
        pltpu.VMEM((EPW,), jnp.int32),
        pltpu.VMEM((EPW,), jnp.int32),
        pltpu.VMEM((EPW,), jnp.float32),
        pltpu.VMEM((N_PAD,), jnp.float32),
        pltpu.VMEM((N_PAD,), jnp.float32),
        pltpu.VMEM((N_PAD,), jnp.float32),
    ],
)


def _k2c_body(att_hbm, src_hbm, m32_hbm, a_hbm,
              attA_v, attB_v, srcA_v, srcB_v, pA_v, pB_v,
              m_v, s_v, tmp32_v, strip_v, m_sh, s_sh):
    cid = lax.axis_index("c")
    sid = lax.axis_index("s")
    wid = cid * NS + sid
    mir = (1 - cid) * NS + sid

    pltpu.sync_copy(m32_hbm.at[:, pl.ds(sid * STRIP, STRIP)], tmp32_v)

    def red(i, carry):
        acc = tmp32_v[0, pl.ds(i * L, L)]
        for r in range(1, NW):
            acc = jnp.maximum(acc, tmp32_v[r, pl.ds(i * L, L)])
        strip_v[pl.ds(i * L, L)] = acc
        return carry

    lax.fori_loop(0, STRIP // L, red, 0)
    pltpu.sync_copy(strip_v, m_sh.at[pl.ds(sid * STRIP, STRIP)])

    def zer(i, carry):
        strip_v[pl.ds(i * L, L)] = jnp.zeros((L,), jnp.float32)
        return carry

    lax.fori_loop(0, STRIP // L, zer, 0)
    pltpu.sync_copy(strip_v, s_sh.at[pl.ds(sid * STRIP, STRIP)])
    plsc.subcore_barrier()
    pltpu.sync_copy(m_sh, m_v)

    for att_v, src_v, p_v, owner in (
        (attA_v, srcA_v, pA_v, wid),
        (attB_v, srcB_v, pB_v, mir),
    ):
        base = pl.multiple_of(owner * EPW, 512)
        pltpu.sync_copy(att_hbm.at[pl.ds(base, EPW)], att_v)
        pltpu.sync_copy(src_hbm.at[pl.ds(base, EPW)], src_v)

        def pexp(i, carry, att_v=att_v, src_v=src_v, p_v=p_v):
            s16 = src_v[pl.ds(i * L, L)]
            mv = plsc.load_gather(m_v, [s16])
            p_v[pl.ds(i * L, L)] = jnp.exp(att_v[pl.ds(i * L, L)] - mv)
            return carry

        lax.fori_loop(0, EPW // L, pexp, 0)

    pltpu.sync_copy(pA_v, s_sh.at[srcA_v], add=True)
    pltpu.sync_copy(pB_v, s_sh.at[srcB_v], add=True)
    plsc.subcore_barrier()
    pltpu.sync_copy(s_sh, s_v)

    def norm(i, carry):
        s16 = srcA_v[pl.ds(i * L, L)]
        sv = plsc.load_gather(s_v, [s16])
        attA_v[pl.ds(i * L, L)] = pA_v[pl.ds(i * L, L)] / jnp.maximum(sv, 1e-16)
        return carry

    lax.fori_loop(0, EPW // L, norm, 0)
    base = pl.multiple_of(wid * EPW, 512)
    pltpu.sync_copy(attA_v, a_hbm.at[pl.ds(base, EPW)])


_k2c = pl.kernel(
    _k2c_body,
    out_type=jax.ShapeDtypeStruct((E_PAD,), jnp.float32),
    mesh=_mesh,
    compiler_params=pltpu.CompilerParams(needs_layout_passes=False),
    scratch_types=[
        pltpu.VMEM((EPW,), jnp.float32),
        pltpu.VMEM((EPW,), jnp.float32),
        pltpu.VMEM((EPW,), jnp.int32),
        pltpu.VMEM((EPW,), jnp.int32),
        pltpu.VMEM((EPW,), jnp.float32),
        pltpu.VMEM((EPW,), jnp.float32),
        pltpu.VMEM((N_PAD,), jnp.float32),
        pltpu.VMEM((N_PAD,), jnp.float32),
        pltpu.VMEM((NW, STRIP), jnp.float32),
        pltpu.VMEM((STRIP,), jnp.float32),
        pltpu.VMEM_SHARED((N_PAD,), jnp.float32),
        pltpu.VMEM_SHARED((N_PAD,), jnp.float32),
    ],
)


EC = E_PAD // NC
SCAN = 4096
NSC = EC // SCAN
DR = 256
CAP = SCAN + 4 * L
ZROWS = STRIP + 8


def _spmm_body(table_hbm, a_hbm, src_hbm, dst_hbm, zp_hbm,
               srcb, dstb, ab,
               cdst_v, crow_v, ca_v, rows_v, z_v,
               rsem):
    cid = lax.axis_index("c")
    sid = lax.axis_index("s")
    lo = sid * STRIP
    hi = lo + STRIP
    ebase = pl.multiple_of(cid * EC, 4096)

    def zer(i, carry):
        z_v[i // 4, pl.ds((i % 4) * L, L)] = jnp.zeros((L,), jnp.float32)
        return carry

    lax.fori_loop(0, ZROWS * 4, zer, 0)

    def ri(i, carry):
        cdst_v[pl.ds(i * L, L)] = jnp.zeros((L,), jnp.int32)
        crow_v[pl.ds(i * L, L)] = jnp.full((L,), STRIP, jnp.int32)
        ca_v[pl.ds(i * L, L)] = jnp.zeros((L,), jnp.float32)
        return carry

    lax.fori_loop(0, CAP // L, ri, 0)

    def chunk(t, carry):
        base = ebase + t * SCAN
        pltpu.sync_copy(src_hbm.at[pl.ds(base, SCAN)], srcb)
        pltpu.sync_copy(dst_hbm.at[pl.ds(base, SCAN)], dstb)
        pltpu.sync_copy(a_hbm.at[pl.ds(base, SCAN)], ab)

        def sv(i, c):
            s16 = srcb[pl.ds(i * L, L)]
            m = jnp.logical_and(s16 >= lo, s16 < hi)
            plsc.store_compressed(cdst_v.at[pl.ds(c, L)], dstb[pl.ds(i * L, L)], mask=m)
            plsc.store_compressed(crow_v.at[pl.ds(c, L)], s16 - lo, mask=m)
            plsc.store_compressed(ca_v.at[pl.ds(c, L)], ab[pl.ds(i * L, L)], mask=m)
            return c + plsc.all_reduce_population_count(m)[0]

        c = lax.fori_loop(0, SCAN // L, sv, jnp.int32(0))

        nch = (c + DR - 1) // DR

        def dk(k, carry2):
            pltpu.async_copy(
                table_hbm.at[cdst_v.at[pl.ds(k * DR, DR)]], rows_v, rsem
            ).wait()

            def pe(jv, carry3, k=k):
                r16 = crow_v[pl.ds(k * DR + jv * L, L)]
                a16 = ca_v[pl.ds(k * DR + jv * L, L)]
                for jj in range(L):
                    r = r16[jj]
                    av = a16[jj]
                    row = jv * L + jj
                    for q in range(OUT_DIM // L):
                        z_v[r, pl.ds(q * L, L)] = (
                            z_v[r, pl.ds(q * L, L)]
                            + rows_v[row, pl.ds(q * L, L)] * av
                        )
                return carry3

            lax.fori_loop(0, DR // L, pe, 0)
            return carry2

        lax.fori_loop(0, nch, dk, 0)
        lax.fori_loop(0, CAP // L, ri, 0)
        return carry

    lax.fori_loop(0, NSC, chunk, 0)

    pltpu.sync_copy(
        z_v.at[pl.ds(0, STRIP)], zp_hbm.at[cid, pl.ds(lo, STRIP)]
    )


_spmm = pl.kernel(
    _spmm_body,
    out_type=jax.ShapeDtypeStruct((NC, N_PAD, OUT_DIM), jnp.float32),
    mesh=_mesh,
    compiler_params=pltpu.CompilerParams(
        needs_layout_passes=False, use_tc_tiling_on_sc=False),
    scratch_types=[
        pltpu.VMEM((SCAN,), jnp.int32),
        pltpu.VMEM((SCAN,), jnp.int32),
        pltpu.VMEM((SCAN,), jnp.float32),
        pltpu.VMEM((CAP,), jnp.int32),
        pltpu.VMEM((CAP,), jnp.int32),
        pltpu.VMEM((CAP,), jnp.float32),
        pltpu.VMEM((DR, OUT_DIM), jnp.float32),
        pltpu.VMEM((ZROWS, OUT_DIM), jnp.float32),
        pltpu.SemaphoreType.DMA,
    ],
)


@jax.jit
def kernel(features, edge_index, W0, W1, att_w):
    src = edge_index[0]
    dst = edge_index[1]
    extra = E_PAD - E
    pad = (N + (jnp.arange(extra, dtype=jnp.int32) % (N_PAD - N))).astype(jnp.int32)
    srcp = jnp.concatenate([src, pad])
    dstp = jnp.concatenate([dst, pad])

    xp = jnp.pad(features, ((0, N_PAD - N), (0, 0)))
    wa = jnp.zeros((OUT_DIM, 128), jnp.float32)
    wa = wa.at[:, 0].set(att_w[:OUT_DIM, 0])
    wa = wa.at[:, 1].set(att_w[OUT_DIM + 1 : 2 * OUT_DIM + 1, 0])

    h, A = _dense(xp, W0, W1, wa)
    als = A[:, 0]
    ald = A[:, 1]

    att, m32 = _k2a(srcp, dstp, als, ald)
    a = _k2c(att, srcp, m32)

    z1p = _spmm(h, a, srcp, dstp)
    z1 = _add2(z1p[0], z1p[1], N_PAD, 1024)
    z2p = _spmm(z1, a, srcp, dstp)
    out = _add2(z2p[0, :N], z2p[1, :N], N, 1000)
    return out


# speedup vs baseline: 12.9654x; 12.9654x over previous
import functools

import jax
import jax.numpy as jnp
from jax import lax
from jax.experimental import pallas as pl
from jax.experimental.pallas import tpu as pltpu
from jax.experimental.pallas import tpu_sc as plsc

N = 10000
E = 160000
IN_DIM = 256
HIDDEN = 128
OUT_DIM = 64

NC = 2
NS = 16
NW = NC * NS
L = 16

N_PAD = 10240
STRIP = N_PAD // NS
E_PAD = 163840
EPW = E_PAD // NW
CH = 512
CHUNKS = EPW // CH
NEG = -3.0e38

_mesh = plsc.VectorSubcoreMesh(
    core_axis_name="c", subcore_axis_name="s", num_cores=NC, num_subcores=NS
)


def _dup_safe_max(m_ref, idx, val):
    def cond(mask):
        return jnp.any(mask)

    def body(mask):
        cur = plsc.load_gather(m_ref, [idx], mask=mask)
        need = jnp.logical_and(mask, val > cur)
        plsc.store_scatter(m_ref, [idx], val, mask=need)
        back = plsc.load_gather(m_ref, [idx], mask=need)
        return jnp.logical_and(need, back < val)

    lax.while_loop(cond, body, jnp.ones((L,), jnp.bool_))


def _dense_body(x_ref, w0_ref, w1_ref, wa_ref, h_ref, al_ref):
    h1 = jnp.maximum(
        jnp.dot(x_ref[...], w0_ref[...], preferred_element_type=jnp.float32), 0.0
    )
    h2 = jnp.dot(h1, w1_ref[...], preferred_element_type=jnp.float32)
    h_ref[...] = h2
    al_ref[...] = jnp.dot(h2, wa_ref[...], preferred_element_type=jnp.float32)


def _dense(xp, W0, W1, wa):
    return pl.pallas_call(
        _dense_body,
        grid=(10,),
        in_specs=[
            pl.BlockSpec((N_PAD // 10, IN_DIM), lambda i: (i, 0)),
            pl.BlockSpec((IN_DIM, HIDDEN), lambda i: (0, 0)),
            pl.BlockSpec((HIDDEN, OUT_DIM), lambda i: (0, 0)),
            pl.BlockSpec((OUT_DIM, 128), lambda i: (0, 0)),
        ],
        out_specs=[
            pl.BlockSpec((N_PAD // 10, OUT_DIM), lambda i: (i, 0)),
            pl.BlockSpec((N_PAD // 10, 128), lambda i: (i, 0)),
        ],
        out_shape=[
            jax.ShapeDtypeStruct((N_PAD, OUT_DIM), jnp.float32),
            jax.ShapeDtypeStruct((N_PAD, 128), jnp.float32),
        ],
    )(xp, W0, W1, wa)


def _add_body(a_ref, b_ref, o_ref):
    o_ref[...] = a_ref[...] + b_ref[...]


def _add2(a, b, rows, blk):
    return pl.pallas_call(
        _add_body,
        grid=(rows // blk,),
        in_specs=[
            pl.BlockSpec((blk, OUT_DIM), lambda i: (i, 0)),
            pl.BlockSpec((blk, OUT_DIM), lambda i: (i, 0)),
        ],
        out_specs=pl.BlockSpec((blk, OUT_DIM), lambda i: (i, 0)),
        out_shape=jax.ShapeDtypeStruct((rows, OUT_DIM), jnp.float32),
    )(a, b)


def _k2a_body(src_hbm, dst_hbm, als_hbm, ald_hbm, att_hbm, m32_hbm,
              src_v, dst_v, att_v, als_v, ald_v, m_v):
    cid = lax.axis_index("c")
    sid = lax.axis_index("s")
    wid = sid * NC + cid
    base = pl.multiple_of(wid * EPW, 512)
    pltpu.sync_copy(src_hbm.at[pl.ds(base, EPW)], src_v)
    pltpu.sync_copy(dst_hbm.at[pl.ds(base, EPW)], dst_v)
    pltpu.sync_copy(als_hbm, als_v)
    pltpu.sync_copy(ald_hbm, ald_v)

    def init(i, carry):
        m_v[pl.ds(i * L, L)] = jnp.full((L,), NEG, jnp.float32)
        return carry

    lax.fori_loop(0, N_PAD // L, init, 0)

    def edge(i, carry):
        s16 = src_v[pl.ds(i * L, L)]
        d16 = dst_v[pl.ds(i * L, L)]
        av = plsc.load_gather(als_v, [s16]) + plsc.load_gather(ald_v, [d16])
        att = jnp.maximum(av, 0.01 * av)
        att_v[pl.ds(i * L, L)] = att
        _dup_safe_max(m_v, s16, att)
        return carry

    lax.fori_loop(0, EPW // L, edge, 0)
    pltpu.sync_copy(att_v, att_hbm.at[pl.ds(base, EPW)])
    pltpu.sync_copy(m_v, m32_hbm.at[wid])


_k2a = pl.kernel(
    _k2a_body,
    out_type=[
        jax.ShapeDtypeStruct((E_PAD,), jnp.float32),
        jax.ShapeDtypeStruct((NW, N_PAD), jnp.float32),
    ],
    mesh=_mesh,
    compiler_params=pltpu.CompilerParams(needs_layout_passes=False),
    scratch_types=[
        pltpu.VMEM((EPW,), jnp.int32),
        pltpu.VMEM((EPW,), jnp.int32),
        pltpu.VMEM((EPW,), jnp.float32),
        pltpu.VMEM((N_PAD,), jnp.float32),
        pltpu.VMEM((N_PAD,), jnp.float32),
        pltpu.VMEM((N_PAD,), jnp.float32),
    ],
)


def _k2c_body(att_hbm, src_hbm, m32_hbm, a_hbm,
              attA_v, attB_v, srcA_v, srcB_v, pA_v, pB_v,
              m_v, s_v, tmp32_v, strip_v, m_sh, s_sh):
    cid = lax.axis_index("c")
    sid = lax.axis_index("s")
    wid = sid * NC + cid
    mir = sid * NC + (1 - cid)

    pltpu.sync_copy(m32_hbm.at[:, pl.ds(sid * STRIP, STRIP)], tmp32_v)

    def red(i, carry):
        acc = tmp32_v[0, pl.ds(i * L, L)]
        for r in range(1, NW):
            acc = jnp.maximum(acc, tmp32_v[r, pl.ds(i * L, L)])
        strip_v[pl.ds(i * L, L)] = acc
        return carry

    lax.fori_loop(0, STRIP // L, red, 0)
    pltpu.sync_copy(strip_v, m_sh.at[pl.ds(sid * STRIP, STRIP)])

    def zer(i, carry):
        strip_v[pl.ds(i * L, L)] = jnp.zeros((L,), jnp.float32)
        return carry

    lax.fori_loop(0, STRIP // L, zer, 0)
    pltpu.sync_copy(strip_v, s_sh.at[pl.ds(sid * STRIP, STRIP)])
    plsc.subcore_barrier()
    pltpu.sync_copy(m_sh, m_v)

    for att_v, src_v, p_v, owner in (
        (attA_v, srcA_v, pA_v, wid),
        (attB_v, srcB_v, pB_v, mir),
    ):
        base = pl.multiple_of(owner * EPW, 512)
        pltpu.sync_copy(att_hbm.at[pl.ds(base, EPW)], att_v)
        pltpu.sync_copy(src_hbm.at[pl.ds(base, EPW)], src_v)

        def pexp(i, carry, att_v=att_v, src_v=src_v, p_v=p_v):
            s16 = src_v[pl.ds(i * L, L)]
            mv = plsc.load_gather(m_v, [s16])
            p_v[pl.ds(i * L, L)] = jnp.exp(att_v[pl.ds(i * L, L)] - mv)
            return carry

        lax.fori_loop(0, EPW // L, pexp, 0)

    pltpu.sync_copy(pA_v, s_sh.at[srcA_v], add=True)
    pltpu.sync_copy(pB_v, s_sh.at[srcB_v], add=True)
    plsc.subcore_barrier()
    pltpu.sync_copy(s_sh, s_v)

    def norm(i, carry):
        s16 = srcA_v[pl.ds(i * L, L)]
        sv = plsc.load_gather(s_v, [s16])
        attA_v[pl.ds(i * L, L)] = pA_v[pl.ds(i * L, L)] / jnp.maximum(sv, 1e-16)
        return carry

    lax.fori_loop(0, EPW // L, norm, 0)
    base = pl.multiple_of(wid * EPW, 512)
    pltpu.sync_copy(attA_v, a_hbm.at[pl.ds(base, EPW)])


_k2c = pl.kernel(
    _k2c_body,
    out_type=jax.ShapeDtypeStruct((E_PAD,), jnp.float32),
    mesh=_mesh,
    compiler_params=pltpu.CompilerParams(needs_layout_passes=False),
    scratch_types=[
        pltpu.VMEM((EPW,), jnp.float32),
        pltpu.VMEM((EPW,), jnp.float32),
        pltpu.VMEM((EPW,), jnp.int32),
        pltpu.VMEM((EPW,), jnp.int32),
        pltpu.VMEM((EPW,), jnp.float32),
        pltpu.VMEM((EPW,), jnp.float32),
        pltpu.VMEM((N_PAD,), jnp.float32),
        pltpu.VMEM((N_PAD,), jnp.float32),
        pltpu.VMEM((NW, STRIP), jnp.float32),
        pltpu.VMEM((STRIP,), jnp.float32),
        pltpu.VMEM_SHARED((N_PAD,), jnp.float32),
        pltpu.VMEM_SHARED((N_PAD,), jnp.float32),
    ],
)


def _make_spmm(ntab, CH, CHUNKS):
    def body(table_hbm, a_hbm, src3_hbm, dst3_hbm, dst3b_hbm, zp_hbm,
             a_v, src_v, dst_v, dst2_v, rows_v, rows2_v, zrow_v, z_sh,
             gsem0, gsem1, ssem0, ssem1):
        cid = lax.axis_index("c")
        sid = lax.axis_index("s")
        wid = sid * NC + cid
        base = pl.multiple_of(wid * EPW, 512)
        pltpu.sync_copy(a_hbm.at[pl.ds(base, EPW)], a_v)
        pltpu.sync_copy(src3_hbm.at[wid], src_v)
        pltpu.sync_copy(dst3_hbm.at[wid], dst_v)
        if ntab == 2:
            pltpu.sync_copy(dst3b_hbm.at[wid], dst2_v)

        def zer(i, carry):
            zrow_v[i // 4, pl.ds((i % 4) * L, L)] = jnp.zeros((L,), jnp.float32)
            return carry

        lax.fori_loop(0, 40 * 4, zer, 0)

        def zcp(k, carry):
            pltpu.sync_copy(zrow_v, z_sh.at[pl.ds(sid * STRIP + k * 40, 40)])
            return carry

        lax.fori_loop(0, STRIP // 40, zcp, 0)
        plsc.subcore_barrier()

        gsems = (gsem0, gsem1)
        ssems = (ssem0, ssem1)
        gd = {}
        gd2 = {}
        sd = {}

        def fire_gather(c):
            b = c % 2
            gd[b] = pltpu.async_copy(
                table_hbm.at[dst_v.at[c]], rows_v.at[b], gsems[b])
            if ntab == 2:
                gd2[b] = pltpu.async_copy(
                    table_hbm.at[dst2_v.at[c]], rows2_v.at[b], gsems[b])

        def scale(b, c):
            def sc16(e, carry):
                for j in range(L):
                    eidx = c * CH + e * L + j
                    av = plsc.load_gather(
                        a_v, [jnp.full((L,), 0, jnp.int32) + eidx])
                    row = e * L + j
                    for q in range(OUT_DIM // L):
                        x = rows_v[b, row, pl.ds(q * L, L)]
                        if ntab == 2:
                            x = x + rows2_v[b, row, pl.ds(q * L, L)]
                        rows_v[b, row, pl.ds(q * L, L)] = x * av
                return carry

            lax.fori_loop(0, CH // L, sc16, 0)

        fire_gather(0)
        for c in range(CHUNKS):
            b = c % 2
            gd[b].wait()
            if ntab == 2:
                gd2[b].wait()
            if c + 1 < CHUNKS:
                if c >= 1:
                    sd[1 - b].wait()
                fire_gather(c + 1)
            scale(b, c)
            sd[b] = pltpu.async_copy(
                rows_v.at[b], z_sh.at[src_v.at[c]], ssems[b], add=True)
        sd[0].wait()
        sd[1].wait()

        plsc.subcore_barrier()
        pltpu.sync_copy(
            z_sh.at[pl.ds(sid * STRIP, STRIP)],
            zp_hbm.at[cid, pl.ds(sid * STRIP, STRIP)],
        )

    return pl.kernel(
        body,
        out_type=jax.ShapeDtypeStruct((NC, N_PAD, OUT_DIM), jnp.float32),
        mesh=_mesh,
        compiler_params=pltpu.CompilerParams(
            needs_layout_passes=False, use_tc_tiling_on_sc=False),
        scratch_types=[
            pltpu.VMEM((EPW,), jnp.float32),
            pltpu.VMEM((CHUNKS, CH), jnp.int32),
            pltpu.VMEM((CHUNKS, CH), jnp.int32),
            pltpu.VMEM((CHUNKS, CH) if ntab == 2 else (1, 8), jnp.int32),
            pltpu.VMEM((2, CH, OUT_DIM), jnp.float32),
            pltpu.VMEM((2, CH, OUT_DIM) if ntab == 2 else (1, 8, OUT_DIM),
                       jnp.float32),
            pltpu.VMEM((40, OUT_DIM), jnp.float32),
            pltpu.VMEM_SHARED((N_PAD, OUT_DIM), jnp.float32),
            pltpu.SemaphoreType.DMA,
            pltpu.SemaphoreType.DMA,
            pltpu.SemaphoreType.DMA,
            pltpu.SemaphoreType.DMA,
        ],
    )


_spmm1 = _make_spmm(1, 512, 10)
_spmm2 = _make_spmm(2, 256, 20)


@jax.jit
def kernel(features, edge_index, W0, W1, att_w):
    src = edge_index[0]
    dst = edge_index[1]
    extra = E_PAD - E
    pad = (N + (jnp.arange(extra, dtype=jnp.int32) % (N_PAD - N))).astype(jnp.int32)
    srcp = jnp.concatenate([src, pad])
    dstp = jnp.concatenate([dst, pad])

    xp = jnp.pad(features, ((0, N_PAD - N), (0, 0)))
    wa = jnp.zeros((OUT_DIM, 128), jnp.float32)
    wa = wa.at[:, 0].set(att_w[:OUT_DIM, 0])
    wa = wa.at[:, 1].set(att_w[OUT_DIM + 1 : 2 * OUT_DIM + 1, 0])

    h, A = _dense(xp, W0, W1, wa)
    als = A[:, 0]
    ald = A[:, 1]

    att, m32 = _k2a(srcp, dstp, als, ald)
    a = _k2c(att, srcp, m32)

    src3a = srcp.reshape(NW, 10, 512)
    dst3a = dstp.reshape(NW, 10, 512)
    src3b = srcp.reshape(NW, 20, 256)
    dst3b = dstp.reshape(NW, 20, 256)
    dst3b2 = dst3b + N_PAD

    z1p = _spmm1(h, a, src3a, dst3a, dst3a)
    z2p = _spmm2(z1p.reshape(NC * N_PAD, OUT_DIM), a, src3b, dst3b, dst3b2)
    out = _add2(z2p[0, :N], z2p[1, :N], N, 1000)
    return out
